# baseline (speedup 1.0000x reference)
import functools

import jax
import jax.numpy as jnp
from jax import lax
from jax.experimental import pallas as pl
from jax.experimental.pallas import tpu as pltpu
from jax.experimental.pallas import tpu_sc as plsc

_V = 1000000
_DIM = 64
_PAD = 128
_B = 4096 * 200
_NC = 2
_NS = 16
_NW = _NC * _NS
_BPW = _B // _NW
_CHUNK = 320
_NCHUNK = _BPW // _CHUNK

_W = 256
_NWIN = 999936 // _W
_WPT = 124
_TAIL_OFF = _NWIN * _W
_TAIL = _V - _TAIL_OFF

_mesh = plsc.VectorSubcoreMesh(core_axis_name="c", subcore_axis_name="s")


@functools.partial(
    pl.kernel,
    mesh=_mesh,
    out_type=jax.ShapeDtypeStruct((_V, _PAD), jnp.float32),
    scratch_types=[
        pltpu.VMEM((2, _DIM, _W), jnp.float32),
        pltpu.VMEM((2, _W, _PAD), jnp.float32),
        pltpu.VMEM((_DIM, _TAIL), jnp.float32),
        pltpu.VMEM((_TAIL, _PAD), jnp.float32),
        pltpu.SemaphoreType.DMA,
        pltpu.SemaphoreType.DMA,
        pltpu.SemaphoreType.DMA,
        pltpu.SemaphoreType.DMA,
    ],
    compiler_params=pltpu.CompilerParams(needs_layout_passes=False),
)
def _transpose_k(tf_hbm, out_hbm, in_v, out_v, in_t, out_t, i0, i1, o0, o1):
    wid = lax.axis_index("s") * _NC + lax.axis_index("c")
    base = (wid * _NWIN) // _NW

    isem = (i0, i1)
    osem = (o0, o1)

    def win_off(k):
        return jnp.clip(base + k, 0, _NWIN - 1) * _W

    def start_in(k, b):
        pltpu.async_copy(tf_hbm.at[:, pl.ds(win_off(k), _W)], in_v.at[b], isem[b])

    def wait_in(k, b):
        pltpu.make_async_copy(
            tf_hbm.at[:, pl.ds(win_off(k), _W)], in_v.at[b], isem[b]).wait()

    def start_out(k, b):
        pltpu.async_copy(
            out_v.at[b], out_hbm.at[pl.ds(win_off(k), _W)], osem[b])

    def wait_out(k, b):
        pltpu.make_async_copy(
            out_v.at[b], out_hbm.at[pl.ds(win_off(k), _W)], osem[b]).wait()

    def transpose_win(b, n):
        def trans_body(v, c2):
            for fb in range(_DIM // 16):
                rows = lax.iota(jnp.int32, 16) + (16 * fb)
                cols = jnp.full((16,), v, jnp.int32)
                out_v[b, v, pl.ds(16 * fb, 16)] = plsc.load_gather(
                    in_v.at[b], [rows, cols])
            return c2

        lax.fori_loop(0, n, trans_body, 0)

    start_in(0, 0)
    start_in(1, 1)

    def body(i, carry):
        for b in (0, 1):
            k = 2 * i + b
            wait_in(k, b)
            transpose_win(b, _W)
            pl.when(k >= 2)(lambda: wait_out(k - 2, b))
            start_out(k, b)
            pl.when(k + 2 < _WPT)(lambda: start_in(k + 2, b))
        return carry

    lax.fori_loop(0, _WPT // 2, body, 0)
    for b in (0, 1):
        wait_out(_WPT - 2 + b, b)

    @pl.when(wid == _NW - 1)
    def _():
        pltpu.sync_copy(tf_hbm.at[:, pl.ds(_TAIL_OFF, _TAIL)], in_t)

        def trans_tail(v, c2):
            for fb in range(_DIM // 16):
                rows = lax.iota(jnp.int32, 16) + (16 * fb)
                cols = jnp.full((16,), v, jnp.int32)
                out_t[v, pl.ds(16 * fb, 16)] = plsc.load_gather(in_t, [rows, cols])
            return c2

        lax.fori_loop(0, _TAIL, trans_tail, 0)
        pltpu.sync_copy(out_t, out_hbm.at[pl.ds(_TAIL_OFF, _TAIL)])


@functools.partial(
    pl.kernel,
    mesh=_mesh,
    out_type=jax.ShapeDtypeStruct((_B, _PAD), jnp.float32),
    scratch_types=[
        pltpu.VMEM((_BPW,), jnp.int32),
        pltpu.VMEM((2, _CHUNK, _PAD), jnp.float32),
        pltpu.SemaphoreType.DMA,
        pltpu.SemaphoreType.DMA,
        pltpu.SemaphoreType.DMA,
        pltpu.SemaphoreType.DMA,
    ],
)
def _emb_lookup(idx_hbm, table_hbm, out_hbm, idx_v, rows_v, g0, g1, o0, o1):
    wid = lax.axis_index("s") * _NC + lax.axis_index("c")
    base = wid * _BPW
    pltpu.sync_copy(idx_hbm.at[pl.ds(base, _BPW)], idx_v)

    gsem = (g0, g1)
    osem = (o0, o1)

    def idx_slice(c):
        return idx_v.at[pl.ds(c * _CHUNK, _CHUNK)]

    def out_slice(c):
        return out_hbm.at[pl.ds(base + c * _CHUNK, _CHUNK)]

    def start_gather(c, b):
        pltpu.async_copy(table_hbm.at[idx_slice(c)], rows_v.at[b], gsem[b])

    def wait_gather(c, b):
        pltpu.make_async_copy(table_hbm.at[idx_slice(c)], rows_v.at[b], gsem[b]).wait()

    def start_write(c, b):
        pltpu.async_copy(rows_v.at[b], out_slice(c), osem[b])

    def wait_write(c, b):
        pltpu.make_async_copy(rows_v.at[b], out_slice(c), osem[b]).wait()

    start_gather(0, 0)
    start_gather(1, 1)

    def body(i, carry):
        for b in (0, 1):
            cc = 2 * i + b
            wait_gather(cc, b)
            start_write(cc, b)
            wait_write(cc, b)
            start_gather(cc + 2, b)
        return carry

    lax.fori_loop(0, _NCHUNK // 2 - 1, body, 0)

    for b in (0, 1):
        wait_gather(_NCHUNK - 2 + b, b)
        start_write(_NCHUNK - 2 + b, b)
    for b in (0, 1):
        wait_write(_NCHUNK - 2 + b, b)


def kernel(x, table):
    td = _transpose_k(table.T)
    out = _emb_lookup(x.reshape(-1), td)
    return out[:, :_DIM].reshape(x.shape + (table.shape[1],))


# speedup vs baseline: 1.9689x; 1.9689x over previous
import functools

import jax
import jax.numpy as jnp
from jax import lax
from jax.experimental import pallas as pl
from jax.experimental.pallas import tpu as pltpu
from jax.experimental.pallas import tpu_sc as plsc

_DIM = 64
_PAD = 128
_B = 4096 * 200
_NC = 2
_NS = 16
_NW = _NC * _NS
_BPW = _B // _NW
_CHUNK = 400
_NCHUNK = _BPW // _CHUNK

_mesh = plsc.VectorSubcoreMesh(core_axis_name="c", subcore_axis_name="s")


@functools.partial(
    pl.kernel,
    mesh=_mesh,
    out_type=jax.ShapeDtypeStruct((_B, _PAD), jnp.float32),
    scratch_types=[
        pltpu.VMEM((_CHUNK,), jnp.int32),
        pltpu.VMEM((_CHUNK,), jnp.int32),
        pltpu.VMEM((2, _CHUNK, _PAD), jnp.float32),
        pltpu.SemaphoreType.DMA,
        pltpu.SemaphoreType.DMA,
        pltpu.SemaphoreType.DMA,
        pltpu.SemaphoreType.DMA,
        pltpu.SemaphoreType.DMA,
        pltpu.SemaphoreType.DMA,
    ],
)
def _emb_lookup(idx_hbm, table_hbm, out_hbm, idx_a, idx_b, rows_v,
                i0, i1, g0, g1, o0, o1):
    idx_bufs = (idx_a, idx_b)
    wid = lax.axis_index("s") * _NC + lax.axis_index("c")
    base = wid * _BPW

    isem = (i0, i1)
    gsem = (g0, g1)
    osem = (o0, o1)

    def idx_src(c):
        return idx_hbm.at[pl.ds(base + c * _CHUNK, _CHUNK)]

    def out_slice(c):
        return out_hbm.at[pl.ds(base + c * _CHUNK, _CHUNK)]

    def start_idx(c, b):
        pltpu.async_copy(idx_src(c), idx_bufs[b], isem[b])

    def wait_idx(c, b):
        pltpu.make_async_copy(idx_src(c), idx_bufs[b], isem[b]).wait()

    def start_gather(b):
        pltpu.async_copy(table_hbm.at[idx_bufs[b]], rows_v.at[b], gsem[b])

    def wait_gather(b):
        pltpu.make_async_copy(table_hbm.at[idx_bufs[b]], rows_v.at[b], gsem[b]).wait()

    def start_write(c, b):
        pltpu.async_copy(rows_v.at[b], out_slice(c), osem[b])

    def wait_write(c, b):
        pltpu.make_async_copy(rows_v.at[b], out_slice(c), osem[b]).wait()

    for b in (0, 1):
        start_idx(b, b)
    for b in (0, 1):
        wait_idx(b, b)
        start_gather(b)

    def body(i, carry):
        for b in (0, 1):
            cc = 2 * i + b
            wait_gather(b)
            start_write(cc, b)
            wait_write(cc, b)
            start_idx(cc + 2, b)
            wait_idx(cc + 2, b)
            start_gather(b)
        return carry

    lax.fori_loop(0, _NCHUNK // 2 - 1, body, 0)

    for b in (0, 1):
        wait_gather(b)
        start_write(_NCHUNK - 2 + b, b)
    for b in (0, 1):
        wait_write(_NCHUNK - 2 + b, b)


def kernel(x, table):
    tp = jnp.pad(table, ((0, 0), (0, _PAD - _DIM)))
    idx = x.reshape(-1)
    out = _emb_lookup(idx, tp)
    return out[:, :_DIM].reshape(x.shape + (table.shape[1],))


# speedup vs baseline: 1.9714x; 1.0013x over previous
import functools

import jax
import jax.numpy as jnp
from jax import lax
from jax.experimental import pallas as pl
from jax.experimental.pallas import tpu as pltpu
from jax.experimental.pallas import tpu_sc as plsc

_DIM = 64
_PAD = 128
_B = 4096 * 200
_NC = 2
_NS = 16
_NW = _NC * _NS
_BPW = _B // _NW
_NBUF = 4
_CHUNK = 200
_NCHUNK = _BPW // _CHUNK

_mesh = plsc.VectorSubcoreMesh(core_axis_name="c", subcore_axis_name="s")


@functools.partial(
    pl.kernel,
    mesh=_mesh,
    out_type=jax.ShapeDtypeStruct((_B, _PAD), jnp.float32),
    scratch_types=(
        [pltpu.VMEM((_CHUNK,), jnp.int32) for _ in range(_NBUF)]
        + [pltpu.VMEM((_NBUF, _CHUNK, _PAD), jnp.float32)]
        + [pltpu.SemaphoreType.DMA for _ in range(3 * _NBUF)]
    ),
)
def _emb_lookup(idx_hbm, table_hbm, out_hbm, *bufs):
    idx_bufs = bufs[:_NBUF]
    rows_v = bufs[_NBUF]
    isem = bufs[_NBUF + 1:_NBUF + 1 + _NBUF]
    gsem = bufs[_NBUF + 1 + _NBUF:_NBUF + 1 + 2 * _NBUF]
    osem = bufs[_NBUF + 1 + 2 * _NBUF:]

    wid = lax.axis_index("s") * _NC + lax.axis_index("c")
    base = wid * _BPW

    def idx_src(c):
        return idx_hbm.at[pl.ds(base + c * _CHUNK, _CHUNK)]

    def out_slice(c):
        return out_hbm.at[pl.ds(base + c * _CHUNK, _CHUNK)]

    def start_idx(c, b):
        pltpu.async_copy(idx_src(c), idx_bufs[b], isem[b])

    def wait_idx(c, b):
        pltpu.make_async_copy(idx_src(c), idx_bufs[b], isem[b]).wait()

    def start_gather(b):
        pltpu.async_copy(table_hbm.at[idx_bufs[b]], rows_v.at[b], gsem[b])

    def wait_gather(b):
        pltpu.make_async_copy(table_hbm.at[idx_bufs[b]], rows_v.at[b], gsem[b]).wait()

    def start_write(c, b):
        pltpu.async_copy(rows_v.at[b], out_slice(c), osem[b])

    def wait_write(c, b):
        pltpu.make_async_copy(rows_v.at[b], out_slice(c), osem[b]).wait()

    for b in range(_NBUF):
        start_idx(b, b)
    for b in range(_NBUF):
        wait_idx(b, b)
        start_gather(b)

    def body(i, carry):
        for b in range(_NBUF):
            cc = _NBUF * i + b
            wait_gather(b)
            start_write(cc, b)
            wait_write(cc, b)
            start_idx(cc + _NBUF, b)
            wait_idx(cc + _NBUF, b)
            start_gather(b)
        return carry

    lax.fori_loop(0, _NCHUNK // _NBUF - 1, body, 0)

    for b in range(_NBUF):
        wait_gather(b)
        start_write(_NCHUNK - _NBUF + b, b)
    for b in range(_NBUF):
        wait_write(_NCHUNK - _NBUF + b, b)


def kernel(x, table):
    tp = jnp.pad(table, ((0, 0), (0, _PAD - _DIM)))
    idx = x.reshape(-1)
    out = _emb_lookup(idx, tp)
    return out[:, :_DIM].reshape(x.shape + (table.shape[1],))
